# Initial kernel scaffold; baseline (speedup 1.0000x reference)
import jax
import jax.numpy as jnp
from jax.experimental import pallas as pl


def kernel(token_ids, weight):
    raise NotImplementedError("write your pallas kernel here")


# speedup vs baseline: 3.3330x; 3.3330x over previous
import functools

import jax
import jax.numpy as jnp
from jax import lax
from jax.experimental import pallas as pl
from jax.experimental.pallas import tpu as pltpu
from jax.experimental.pallas import tpu_sc as plsc

D = 128
CHUNK = 128
NC, NS = 2, 16
NW = NC * NS


def _emb_body(idx_hbm, table_hbm, out_hbm, idx_v, rows_a, rows_b, sem_a, sem_b):
    n = idx_hbm.shape[0] // (NW * CHUNK)
    wid = lax.axis_index("s") * NC + lax.axis_index("c")
    base = wid * n
    pltpu.sync_copy(idx_hbm.at[pl.ds(base * CHUNK, n * CHUNK)], idx_v)

    def gather(j, buf, sem):
        return pltpu.async_copy(
            table_hbm.at[idx_v.at[pl.ds(j * CHUNK, CHUNK)]], buf, sem)

    def scatter(j, buf):
        pltpu.sync_copy(buf, out_hbm.at[pl.ds((base + j) * CHUNK, CHUNK)])

    gather(0, rows_a, sem_a)

    def body(i, carry):
        j = 2 * i
        gather(j + 1, rows_b, sem_b)
        pltpu.make_async_copy(table_hbm.at[pl.ds(0, CHUNK)], rows_a, sem_a).wait()
        scatter(j, rows_a)
        gather(j + 2, rows_a, sem_a)
        pltpu.make_async_copy(table_hbm.at[pl.ds(0, CHUNK)], rows_b, sem_b).wait()
        scatter(j + 1, rows_b)
        return carry

    lax.fori_loop(0, n // 2 - 1, body, 0)

    gather(n - 1, rows_b, sem_b)
    pltpu.make_async_copy(table_hbm.at[pl.ds(0, CHUNK)], rows_a, sem_a).wait()
    scatter(n - 2, rows_a)
    pltpu.make_async_copy(table_hbm.at[pl.ds(0, CHUNK)], rows_b, sem_b).wait()
    scatter(n - 1, rows_b)


@functools.partial(jax.jit, static_argnames=("total_rows",))
def _emb_call(idx2d, weight, total_rows):
    mesh = plsc.VectorSubcoreMesh(core_axis_name="c", subcore_axis_name="s")
    f = pl.kernel(
        _emb_body,
        out_type=jax.ShapeDtypeStruct((total_rows, D), jnp.float32),
        mesh=mesh,
        scratch_types=[
            pltpu.VMEM((total_rows // NW,), jnp.int32),
            pltpu.VMEM((CHUNK, D), jnp.float32),
            pltpu.VMEM((CHUNK, D), jnp.float32),
            pltpu.SemaphoreType.DMA,
            pltpu.SemaphoreType.DMA,
        ],
    )
    return f(idx2d, weight)


def kernel(token_ids, weight):
    b, s = token_ids.shape
    total = b * s
    idx1d = token_ids.astype(jnp.int32).reshape(total)
    out = _emb_call(idx1d, weight, total)
    return out.reshape(b, s, D)


# speedup vs baseline: 3.3380x; 1.0015x over previous
import functools

import jax
import jax.numpy as jnp
from jax import lax
from jax.experimental import pallas as pl
from jax.experimental.pallas import tpu as pltpu
from jax.experimental.pallas import tpu_sc as plsc

D = 128
CHUNK = 128
NBUF = 5
NC, NS = 2, 16
NW = NC * NS


def _emb_body(idx_hbm, table_hbm, out_hbm, idx_v, bufs, gsem, ssem):
    n = idx_hbm.shape[0] // (NW * CHUNK)
    wid = lax.axis_index("s") * NC + lax.axis_index("c")
    base = wid * n
    pltpu.sync_copy(idx_hbm.at[pl.ds(base * CHUNK, n * CHUNK)], idx_v)

    def gather(j, b):
        pltpu.async_copy(
            table_hbm.at[idx_v.at[pl.ds(j * CHUNK, CHUNK)]],
            bufs.at[b], gsem.at[b])

    def wait_g(b):
        pltpu.make_async_copy(
            table_hbm.at[pl.ds(0, CHUNK)], bufs.at[b], gsem.at[b]).wait()

    def scatter(j, b):
        pltpu.async_copy(
            bufs.at[b], out_hbm.at[pl.ds((base + j) * CHUNK, CHUNK)],
            ssem.at[b])

    def wait_s(b):
        pltpu.make_async_copy(
            bufs.at[b], out_hbm.at[pl.ds(0, CHUNK)], ssem.at[b]).wait()

    def visit(j, b, refill, swait):
        wait_g(b)
        scatter(j, b)
        if refill:
            bp = (b + 3) % NBUF
            if swait:
                wait_s(bp)
            gather(j + 3, bp)

    for c in range(3):
        gather(c, c)
    for j in range(NBUF):
        visit(j, j, refill=True, swait=j >= 2)

    def body(i, carry):
        for b in range(NBUF):
            visit(5 * i + b, b, refill=True, swait=True)
        return carry

    lax.fori_loop(1, n // NBUF - 1, body, 0)

    for j in range(n - NBUF, n):
        visit(j, j % NBUF, refill=j + 3 < n, swait=True)
    for b in range(NBUF):
        wait_s(b)


@functools.partial(jax.jit, static_argnames=("total_rows",))
def _emb_call(idx1d, weight, total_rows):
    mesh = plsc.VectorSubcoreMesh(core_axis_name="c", subcore_axis_name="s")
    f = pl.kernel(
        _emb_body,
        out_type=jax.ShapeDtypeStruct((total_rows, D), jnp.float32),
        mesh=mesh,
        scratch_types=[
            pltpu.VMEM((total_rows // NW,), jnp.int32),
            pltpu.VMEM((NBUF, CHUNK, D), jnp.float32),
            pltpu.SemaphoreType.DMA((NBUF,)),
            pltpu.SemaphoreType.DMA((NBUF,)),
        ],
    )
    return f(idx1d, weight)


def kernel(token_ids, weight):
    b, s = token_ids.shape
    total = b * s
    idx1d = token_ids.astype(jnp.int32).reshape(total)
    out = _emb_call(idx1d, weight, total)
    return out.reshape(b, s, D)


# speedup vs baseline: 5.9490x; 1.7822x over previous
import functools

import jax
import jax.numpy as jnp
from jax import lax
from jax.experimental import pallas as pl
from jax.experimental.pallas import tpu as pltpu
from jax.experimental.pallas import tpu_sc as plsc

D = 128
NBUF = 6
LEAD = 4
NC, NS = 2, 16
NW = NC * NS


def _emb_body(idx_hbm, table_hbm, out_hbm, idx_v, bufs, gsem, ssem):
    n = idx_hbm.shape[0] // NW
    s = idx_hbm.shape[1]
    wid = lax.axis_index("s") * NC + lax.axis_index("c")
    base = wid * n
    pltpu.sync_copy(idx_hbm.at[pl.ds(base, n)], idx_v)

    def gather(j, b):
        pltpu.async_copy(table_hbm.at[idx_v.at[j]], bufs.at[b], gsem.at[b])

    def wait_g(b):
        pltpu.make_async_copy(out_hbm.at[0], bufs.at[b], gsem.at[b]).wait()

    def scatter(j, b):
        pltpu.async_copy(bufs.at[b], out_hbm.at[base + j], ssem.at[b])

    def wait_s(b):
        pltpu.make_async_copy(bufs.at[b], out_hbm.at[0], ssem.at[b]).wait()

    def visit(j, b, refill, swait):
        wait_g(b)
        scatter(j, b)
        if refill:
            bp = (b + LEAD) % NBUF
            if swait:
                wait_s(bp)
            gather(j + LEAD, bp)

    for c in range(LEAD):
        gather(c, c)
    for j in range(NBUF):
        visit(j, j, refill=True, swait=j >= NBUF - LEAD)

    def body(i, carry):
        for b in range(NBUF):
            visit(NBUF * i + b, b, refill=True, swait=True)
        return carry

    lax.fori_loop(1, n // NBUF - 1, body, 0)

    for j in range(n - NBUF - n % NBUF, n):
        visit(j, j % NBUF, refill=j + LEAD < n, swait=True)
    for b in range(NBUF):
        wait_s(b)


@functools.partial(jax.jit, static_argnames=("bb", "ss"))
def _emb_call(idx, weight, bb, ss):
    mesh = plsc.VectorSubcoreMesh(core_axis_name="c", subcore_axis_name="s")
    f = pl.kernel(
        _emb_body,
        out_type=jax.ShapeDtypeStruct((bb, ss, D), jnp.float32),
        mesh=mesh,
        scratch_types=[
            pltpu.VMEM((bb // NW, ss), jnp.int32),
            pltpu.VMEM((NBUF, ss, D), jnp.float32),
            pltpu.SemaphoreType.DMA((NBUF,)),
            pltpu.SemaphoreType.DMA((NBUF,)),
        ],
    )
    return f(idx, weight)


def kernel(token_ids, weight):
    b, s = token_ids.shape
    return _emb_call(token_ids.astype(jnp.int32), weight, b, s)


# speedup vs baseline: 10.4117x; 1.7502x over previous
import functools

import jax
import jax.numpy as jnp
from jax import lax
from jax.experimental import pallas as pl
from jax.experimental.pallas import tpu as pltpu
from jax.experimental.pallas import tpu_sc as plsc

D = 128
CHUNK = 128
NBUF = 6
LEAD = 4
NC, NS = 2, 16
NW = NC * NS


def _emb_body(idx_hbm, table_hbm, out_hbm, idx_v, bufs, gsem, ssem):
    n = idx_hbm.shape[0] // (NW * CHUNK)
    wid = lax.axis_index("s") * NC + lax.axis_index("c")
    base = wid * n
    pltpu.sync_copy(idx_hbm.at[pl.ds(base * CHUNK, n * CHUNK)], idx_v)

    def gather(j, b):
        pltpu.async_copy(
            table_hbm.at[idx_v.at[pl.ds(j * CHUNK, CHUNK)]],
            bufs.at[b], gsem.at[b])

    def wait_g(b):
        pltpu.make_async_copy(
            table_hbm.at[pl.ds(0, CHUNK)], bufs.at[b], gsem.at[b]).wait()

    def scatter(j, b):
        pltpu.async_copy(
            bufs.at[b], out_hbm.at[pl.ds((base + j) * CHUNK, CHUNK)],
            ssem.at[b])

    def wait_s(b):
        pltpu.make_async_copy(
            bufs.at[b], out_hbm.at[pl.ds(0, CHUNK)], ssem.at[b]).wait()

    def visit(j, b, refill, swait):
        wait_g(b)
        scatter(j, b)
        if refill:
            bp = (b + LEAD) % NBUF
            if swait:
                wait_s(bp)
            gather(j + LEAD, bp)

    for c in range(LEAD):
        gather(c, c)
    for j in range(NBUF):
        visit(j, j, refill=True, swait=j >= NBUF - LEAD)

    def body(i, carry):
        for b in range(NBUF):
            visit(NBUF * i + b, b, refill=True, swait=True)
        return carry

    lax.fori_loop(1, (n - NBUF - n % NBUF) // NBUF, body, 0)

    for j in range(n - NBUF - n % NBUF, n):
        visit(j, j % NBUF, refill=j + LEAD < n, swait=True)
    for b in range(NBUF):
        wait_s(b)


@functools.partial(jax.jit, static_argnames=("total_rows",))
def _emb_call(idx1d, weight, total_rows):
    mesh = plsc.VectorSubcoreMesh(core_axis_name="c", subcore_axis_name="s")
    f = pl.kernel(
        _emb_body,
        out_type=jax.ShapeDtypeStruct((total_rows, D), jnp.float32),
        mesh=mesh,
        scratch_types=[
            pltpu.VMEM((total_rows // NW,), jnp.int32),
            pltpu.VMEM((NBUF, CHUNK, D), jnp.float32),
            pltpu.SemaphoreType.DMA((NBUF,)),
            pltpu.SemaphoreType.DMA((NBUF,)),
        ],
    )
    return f(idx1d, weight)


def kernel(token_ids, weight):
    b, s = token_ids.shape
    total = b * s
    idx1d = token_ids.astype(jnp.int32).T.reshape(total)
    out = _emb_call(idx1d, weight, total)
    return out.reshape(s, b, D).transpose(1, 0, 2)
